# baseline (speedup 1.0000x reference)
import jax
import jax.numpy as jnp
from jax import lax
from jax.experimental import pallas as pl
from jax.experimental.pallas import tpu as pltpu
from jax.experimental.pallas import tpu_sc as plsc

NROWS = 100000
DIM = 512
NSEG = 64
NCORES = 2
NSUB = 16
COLS = DIM // NCORES
KGRP = COLS // 16
CHUNK = 192
NFULL = NROWS // CHUNK
TAIL = NROWS - NFULL * CHUNK
TAIL0 = NFULL * CHUNK
CHUNKS_PER_SUB = -(-NFULL // NSUB)
NPAIRS = -(-(CHUNKS_PER_SUB + 1) // 2)


def _slow_path(buf, idxb, acc, ngroups):
  def gstep(g, _):
    segv = idxb[pl.ds(g * 16, 16)]
    for l in range(16):
      seg = segv[l]
      r = g * 16 + l
      for k in range(KGRP):
        plsc.addupdate(acc.at[seg, pl.ds(k * 16, 16)],
                       buf[r, pl.ds(k * 16, 16)])
    return None
  lax.fori_loop(0, ngroups, gstep, None)


def _body(nf_hbm, b_hbm, out_hbm, part_hbm, buf0, buf1, idx0, idx1,
          acc, comb, tmp, dsem, isem):
  c = lax.axis_index("c")
  s = lax.axis_index("s")
  col0 = c * COLS
  zero16 = jnp.zeros((16,), jnp.float32)
  bufs = (buf0, buf1)
  idxs = (idx0, idx1)

  def start_load(i, b):
    j = s + i * NSUB

    @pl.when(j < NFULL)
    def _():
      row0 = j * CHUNK
      pltpu.async_copy(b_hbm.at[pl.ds(row0, CHUNK)], idxs[b], isem.at[b])
      pltpu.async_copy(nf_hbm.at[pl.ds(row0, CHUNK), pl.ds(col0, COLS)],
                       bufs[b], dsem.at[b])

  def wait_load(i, b):
    j = s + i * NSUB

    @pl.when(j < NFULL)
    def _():
      pltpu.make_async_copy(b_hbm.at[pl.ds(0, CHUNK)], idxs[b],
                            isem.at[b]).wait()
      pltpu.make_async_copy(nf_hbm.at[pl.ds(0, CHUNK), pl.ds(0, COLS)],
                            bufs[b], dsem.at[b]).wait()

  def compute(i, b):
    j = s + i * NSUB
    buf = bufs[b]
    idxb = idxs[b]

    @pl.when(j < NFULL)
    def _():
      seg0 = idxb[pl.ds(0, 16)][0]
      segL = idxb[pl.ds(CHUNK - 16, 16)][15]

      @pl.when(seg0 == segL)
      def _():
        def rstep(r, carry):
          return tuple(cv + buf[r, pl.ds(k * 16, 16)]
                       for k, cv in enumerate(carry))
        sums = lax.fori_loop(0, CHUNK, rstep,
                             tuple(zero16 for _ in range(KGRP)))
        for k in range(KGRP):
          plsc.addupdate(acc.at[seg0, pl.ds(k * 16, 16)], sums[k])

      @pl.when(seg0 != segL)
      def _():
        _slow_path(buf, idxb, acc, CHUNK // 16)

  start_load(0, 0)
  start_load(1, 1)

  def zrow(r, _):
    for k in range(KGRP):
      acc[r, pl.ds(k * 16, 16)] = zero16
    return None
  lax.fori_loop(0, NSEG, zrow, None)

  def pair(i2, _):
    i0 = i2 * 2
    for b in range(2):
      i = i0 + b
      wait_load(i, b)
      compute(i, b)
      start_load(i + 2, b)
    return None
  lax.fori_loop(0, NPAIRS, pair, None)

  @pl.when(s == NSUB - 1)
  def _():
    pltpu.sync_copy(b_hbm.at[pl.ds(TAIL0, TAIL)], idx0.at[pl.ds(0, TAIL)])
    pltpu.sync_copy(nf_hbm.at[pl.ds(TAIL0, TAIL), pl.ds(col0, COLS)],
                    buf0.at[pl.ds(0, TAIL)])
    _slow_path(buf0, idx0, acc, TAIL // 16)

  slot0 = c * NSUB
  pltpu.sync_copy(acc, part_hbm.at[slot0 + s])
  plsc.subcore_barrier()
  pltpu.sync_copy(part_hbm.at[slot0, pl.ds(4 * s, 4)], comb)

  def cstep(t, _):
    pltpu.sync_copy(part_hbm.at[slot0 + t, pl.ds(4 * s, 4)], tmp)
    for r in range(4):
      for k in range(KGRP):
        plsc.addupdate(comb.at[r, pl.ds(k * 16, 16)],
                       tmp[r, pl.ds(k * 16, 16)])
    return None
  lax.fori_loop(1, NSUB, cstep, None)

  pltpu.sync_copy(comb, out_hbm.at[pl.ds(4 * s, 4), pl.ds(col0, COLS)])


@jax.jit
def _segment_sum_sc(node_feat, batch):
  mesh = plsc.VectorSubcoreMesh(core_axis_name="c", subcore_axis_name="s")
  f = pl.kernel(
      _body,
      out_type=[
          jax.ShapeDtypeStruct((NSEG, DIM), jnp.float32),
          jax.ShapeDtypeStruct((NCORES * NSUB, NSEG, COLS), jnp.float32),
      ],
      mesh=mesh,
      scratch_types=[
          pltpu.VMEM((CHUNK, COLS), jnp.float32),
          pltpu.VMEM((CHUNK, COLS), jnp.float32),
          pltpu.VMEM((CHUNK,), jnp.int32),
          pltpu.VMEM((CHUNK,), jnp.int32),
          pltpu.VMEM((NSEG, COLS), jnp.float32),
          pltpu.VMEM((4, COLS), jnp.float32),
          pltpu.VMEM((4, COLS), jnp.float32),
          pltpu.SemaphoreType.DMA((2,)),
          pltpu.SemaphoreType.DMA((2,)),
      ],
  )
  out, _ = f(node_feat, batch)
  return out


def kernel(node_feat, batch):
  return _segment_sum_sc(node_feat, batch.astype(jnp.int32))


# speedup vs baseline: 1.6147x; 1.6147x over previous
import jax
import jax.numpy as jnp
from jax import lax
from jax.experimental import pallas as pl
from jax.experimental.pallas import tpu as pltpu
from jax.experimental.pallas import tpu_sc as plsc

NROWS = 100000
DIM = 512
NSEG = 64
NCORES = 2
NSUB = 16
COLS = DIM // NCORES
KGRP = COLS // 16
CHUNK = 128
NFULL = NROWS // CHUNK
TAIL = NROWS - NFULL * CHUNK
TAIL0 = NFULL * CHUNK
CHUNKS_PER_SUB = -(-NFULL // NSUB)
NTRIPS = -(-(CHUNKS_PER_SUB + 2) // 3)


def _slow_path(buf, idxb, acc, ngroups):
  def gstep(g, _):
    segv = idxb[pl.ds(g * 16, 16)]
    for l in range(16):
      seg = segv[l]
      r = g * 16 + l
      for k in range(KGRP):
        plsc.addupdate(acc.at[seg, pl.ds(k * 16, 16)],
                       buf[r, pl.ds(k * 16, 16)])
    return None
  lax.fori_loop(0, ngroups, gstep, None)


def _body(nf_hbm, b_hbm, out_hbm, part_hbm, buf0, buf1, buf2, idx0, idx1,
          idx2, acc, comb, tmp, dsem, isem):
  c = lax.axis_index("c")
  s = lax.axis_index("s")
  col0 = c * COLS
  zero16 = jnp.zeros((16,), jnp.float32)
  bufs = (buf0, buf1, buf2)
  idxs = (idx0, idx1, idx2)

  def start_load(i, b):
    j = s + i * NSUB

    @pl.when(j < NFULL)
    def _():
      row0 = j * CHUNK
      pltpu.async_copy(b_hbm.at[pl.ds(row0, CHUNK)], idxs[b], isem.at[b])
      pltpu.async_copy(nf_hbm.at[pl.ds(row0, CHUNK), pl.ds(col0, COLS)],
                       bufs[b], dsem.at[b])

  def wait_load(i, b):
    j = s + i * NSUB

    @pl.when(j < NFULL)
    def _():
      pltpu.make_async_copy(b_hbm.at[pl.ds(0, CHUNK)], idxs[b],
                            isem.at[b]).wait()
      pltpu.make_async_copy(nf_hbm.at[pl.ds(0, CHUNK), pl.ds(0, COLS)],
                            bufs[b], dsem.at[b]).wait()

  def compute(i, b):
    j = s + i * NSUB
    buf = bufs[b]
    idxb = idxs[b]

    @pl.when(j < NFULL)
    def _():
      seg0 = idxb[pl.ds(0, 16)][0]
      segL = idxb[pl.ds(CHUNK - 16, 16)][15]

      def rstep(r, carry):
        return tuple(cv + buf[r, pl.ds(k * 16, 16)]
                     for k, cv in enumerate(carry))

      def run_sum(lo, hi, seg):
        sums = lax.fori_loop(lo, hi, rstep,
                             tuple(zero16 for _ in range(KGRP)))
        for k in range(KGRP):
          plsc.addupdate(acc.at[seg, pl.ds(k * 16, 16)], sums[k])

      @pl.when(seg0 == segL)
      def _():
        run_sum(0, CHUNK, seg0)

      @pl.when(seg0 + 1 == segL)
      def _():
        one = jnp.int32(1)
        zero = jnp.int32(0)
        gfull = zero
        lasts = [idxb[pl.ds(g * 16, 16)][15] for g in range(CHUNK // 16)]
        for lv in lasts:
          gfull = gfull + jnp.where(lv == seg0, one, zero)
        segv = idxb[pl.ds(gfull * 16, 16)]
        inb = zero
        for l in range(16):
          inb = inb + jnp.where(segv[l] == seg0, one, zero)
        p = gfull * 16 + inb
        run_sum(0, p, seg0)
        run_sum(p, CHUNK, segL)

      @pl.when(seg0 + 1 < segL)
      def _():
        _slow_path(buf, idxb, acc, CHUNK // 16)

  start_load(0, 0)
  start_load(1, 1)

  def zrow(r, _):
    for k in range(KGRP):
      acc[r, pl.ds(k * 16, 16)] = zero16
    return None
  lax.fori_loop(0, NSEG, zrow, None)

  def trip(i3, _):
    i0 = i3 * 3
    for b in range(3):
      i = i0 + b
      wait_load(i, b)
      start_load(i + 2, (b + 2) % 3)
      compute(i, b)
    return None
  lax.fori_loop(0, NTRIPS, trip, None)

  @pl.when(s == NSUB - 1)
  def _():
    pltpu.sync_copy(b_hbm.at[pl.ds(TAIL0, TAIL)], idx0.at[pl.ds(0, TAIL)])
    pltpu.sync_copy(nf_hbm.at[pl.ds(TAIL0, TAIL), pl.ds(col0, COLS)],
                    buf0.at[pl.ds(0, TAIL)])
    _slow_path(buf0, idx0, acc, TAIL // 16)

  slot0 = c * NSUB
  pltpu.sync_copy(acc, part_hbm.at[slot0 + s])
  plsc.subcore_barrier()
  pltpu.sync_copy(part_hbm.at[slot0, pl.ds(4 * s, 4)], comb)

  def cstep(t, _):
    pltpu.sync_copy(part_hbm.at[slot0 + t, pl.ds(4 * s, 4)], tmp)
    for r in range(4):
      for k in range(KGRP):
        plsc.addupdate(comb.at[r, pl.ds(k * 16, 16)],
                       tmp[r, pl.ds(k * 16, 16)])
    return None
  lax.fori_loop(1, NSUB, cstep, None)

  pltpu.sync_copy(comb, out_hbm.at[pl.ds(4 * s, 4), pl.ds(col0, COLS)])


@jax.jit
def _segment_sum_sc(node_feat, batch):
  mesh = plsc.VectorSubcoreMesh(core_axis_name="c", subcore_axis_name="s")
  f = pl.kernel(
      _body,
      out_type=[
          jax.ShapeDtypeStruct((NSEG, DIM), jnp.float32),
          jax.ShapeDtypeStruct((NCORES * NSUB, NSEG, COLS), jnp.float32),
      ],
      mesh=mesh,
      scratch_types=[
          pltpu.VMEM((CHUNK, COLS), jnp.float32),
          pltpu.VMEM((CHUNK, COLS), jnp.float32),
          pltpu.VMEM((CHUNK, COLS), jnp.float32),
          pltpu.VMEM((CHUNK,), jnp.int32),
          pltpu.VMEM((CHUNK,), jnp.int32),
          pltpu.VMEM((CHUNK,), jnp.int32),
          pltpu.VMEM((NSEG, COLS), jnp.float32),
          pltpu.VMEM((4, COLS), jnp.float32),
          pltpu.VMEM((4, COLS), jnp.float32),
          pltpu.SemaphoreType.DMA((3,)),
          pltpu.SemaphoreType.DMA((3,)),
      ],
  )
  out, _ = f(node_feat, batch)
  return out


def kernel(node_feat, batch):
  return _segment_sum_sc(node_feat, batch.astype(jnp.int32))
